# baseline (speedup 1.0000x reference)
import functools

import jax
import jax.numpy as jnp
from jax import lax
from jax.experimental import pallas as pl
from jax.experimental.pallas import tpu as pltpu, tpu_sc as plsc

N = 10000
E = 320000
NC = 2
NS = 16
NW = NC * NS
EPT = E // NW
K = 80
C = EPT // K
NBUF = 4
RPT = 640
RLAST = N - (NS - 1) * RPT
RB = 80
DW = 128
MB = 2000


def _init_acc(buf_v, acc, s):
    for t in range(RLAST // RB):
        pltpu.sync_copy(buf_v.at[pl.ds(0, RB)],
                        acc.at[pl.ds(s * RPT + t * RB, RB)])

    @pl.when(s < NS - 1)
    def _():
        for t in range(RLAST // RB, RPT // RB):
            pltpu.sync_copy(buf_v.at[pl.ds(0, RB)],
                            acc.at[pl.ds(s * RPT + t * RB, RB)])


def _copy_out(acc, out_c, s):
    @pl.when(s < NS - 1)
    def _():
        sl = pl.ds(s * RPT, RPT)
        pltpu.sync_copy(acc.at[sl], out_c.at[sl])

    @pl.when(s == NS - 1)
    def _():
        sl = pl.ds((NS - 1) * RPT, RLAST)
        pltpu.sync_copy(acc.at[sl], out_c.at[sl])


@functools.cache
def _sc_deg():
    @functools.partial(
        pl.kernel,
        out_type=jax.ShapeDtypeStruct((NC, N, DW), jnp.float32),
        mesh=plsc.VectorSubcoreMesh(core_axis_name="c", subcore_axis_name="s"),
        scratch_types=[
            pltpu.VMEM((C, K), jnp.int32),
            pltpu.VMEM((K, DW), jnp.float32),
            pltpu.VMEM_SHARED((N, DW), jnp.float32),
        ],
        name="sc_deg",
    )
    def deg_kernel(dst_hbm, out_hbm, idx_v, buf_v, acc):
        c = lax.axis_index("c")
        s = lax.axis_index("s")
        wid = c * NS + s

        def fill(val):
            def body(r, carry):
                for q in range(DW // 16):
                    buf_v[r, pl.ds(q * 16, 16)] = jnp.full((16,), val,
                                                           jnp.float32)
                return carry
            lax.fori_loop(0, K, body, 0)

        fill(0.0)
        _init_acc(buf_v, acc, s)
        plsc.subcore_barrier()

        fill(1.0)
        pltpu.sync_copy(dst_hbm.at[wid], idx_v)

        def chunk(j, carry):
            pltpu.sync_copy(buf_v, acc.at[idx_v.at[j]], add=True)
            return carry
        lax.fori_loop(0, C, chunk, 0)
        plsc.subcore_barrier()

        _copy_out(acc, out_hbm.at[c], s)

    return deg_kernel


@functools.cache
def _sc_prop(D):
    @functools.partial(
        pl.kernel,
        out_type=jax.ShapeDtypeStruct((NC, N, D), jnp.float32),
        mesh=plsc.VectorSubcoreMesh(core_axis_name="c", subcore_axis_name="s"),
        scratch_types=(
            [pltpu.VMEM((2, K), jnp.int32) for _ in range(NBUF)]
            + [pltpu.VMEM((K, D), jnp.float32) for _ in range(NBUF)]
            + [pltpu.VMEM_SHARED((N, D), jnp.float32)]
            + [pltpu.SemaphoreType.DMA for _ in range(NBUF)]
        ),
        name=f"sc_prop{D}",
    )
    def prop(eidx_hbm, u_hbm, out_hbm, *sc):
        idxs = sc[0:NBUF]
        rows = sc[NBUF:2 * NBUF]
        acc = sc[2 * NBUF]
        sems = sc[2 * NBUF + 1:]
        c = lax.axis_index("c")
        s = lax.axis_index("s")
        wid = c * NS + s

        def zbody(r, carry):
            for q in range(D // 16):
                rows[0][r, pl.ds(q * 16, 16)] = jnp.zeros((16,), jnp.float32)
            return carry
        lax.fori_loop(0, K, zbody, 0)
        _init_acc(rows[0], acc, s)
        plsc.subcore_barrier()

        my_eidx = eidx_hbm.at[wid]

        def load_gather(j, b):
            pltpu.sync_copy(my_eidx.at[j], idxs[b])
            pltpu.async_copy(u_hbm.at[idxs[b].at[0]], rows[b], sems[b])

        def drain_scatter(b):
            pltpu.make_async_copy(u_hbm.at[idxs[b].at[0]], rows[b],
                                  sems[b]).wait()
            pltpu.sync_copy(rows[b], acc.at[idxs[b].at[1]], add=True)

        for b in range(NBUF):
            load_gather(b, b)

        def quad(t, carry):
            base = NBUF * t
            for b in range(NBUF):
                drain_scatter(b)

                @pl.when(base + b + NBUF < C)
                def _(b=b):
                    load_gather(base + b + NBUF, b)
            return carry
        lax.fori_loop(0, C // NBUF, quad, 0)
        for b in range(C % NBUF):
            drain_scatter(b)
        plsc.subcore_barrier()

        _copy_out(acc, out_hbm.at[c], s)

    return prop


def _dinv(degp_ref):
    deg = degp_ref[0, :, 0:1] + degp_ref[1, :, 0:1] + 1.0
    return lax.rsqrt(deg)


def _tc1_body(degp_ref, x_ref, w1_ref, u1_ref):
    h = jnp.dot(x_ref[...], w1_ref[...], preferred_element_type=jnp.float32)
    u1_ref[...] = _dinv(degp_ref) * h


def _tc2_body(degp_ref, s1_ref, u1_ref, w2_ref, b1_ref, u2_ref):
    dinv = _dinv(degp_ref)
    h1 = dinv * (s1_ref[0] + s1_ref[1] + u1_ref[...]) + b1_ref[...]
    h1 = jnp.maximum(h1, 0.0)
    u2_ref[...] = dinv * jnp.dot(h1, w2_ref[...], preferred_element_type=jnp.float32)


def _tc3_body(degp_ref, s2_ref, u2_ref, b2_ref, u3_ref):
    dinv = _dinv(degp_ref)
    h2 = dinv * (s2_ref[0] + s2_ref[1] + u2_ref[...]) + b2_ref[...]
    u3_ref[...] = dinv * jnp.maximum(h2, 0.0)


def _tc4_body(degp_ref, s3_ref, u3_ref, wz_ref, bz_ref, z_ref):
    p = _dinv(degp_ref) * (s3_ref[0] + s3_ref[1] + u3_ref[...])
    z_ref[...] = jnp.dot(p, wz_ref[...], preferred_element_type=jnp.float32) + bz_ref[...]


def _degp_spec():
    return pl.BlockSpec((2, MB, DW), lambda i: (0, i, 0))


def _rows(d):
    return pl.BlockSpec((MB, d), lambda i: (i, 0))


def _pair(d):
    return pl.BlockSpec((2, MB, d), lambda i: (0, i, 0))


def _full(shape):
    return pl.BlockSpec(shape, lambda i: tuple(0 for _ in shape))


def _tc_call(body, in_specs, out_d, interpret=False):
    return pl.pallas_call(
        body,
        grid=(N // MB,),
        in_specs=in_specs,
        out_specs=_rows(out_d),
        out_shape=jax.ShapeDtypeStruct((N, out_d), jnp.float32),
        interpret=interpret,
    )


def _tc1(degp, x, W1, interpret=False):
    return _tc_call(_tc1_body, [_degp_spec(), _rows(128), _full((128, 128))],
                    128, interpret)(degp, x, W1)


def _tc2(degp, s1, u1, W2p, b1r, interpret=False):
    return _tc_call(_tc2_body,
                    [_degp_spec(), _pair(128), _rows(128), _full((128, 128)),
                     _full((1, 128))], 128, interpret)(degp, s1, u1, W2p, b1r)


def _tc3(degp, s2, u2, b2r, interpret=False):
    return _tc_call(_tc3_body,
                    [_degp_spec(), _pair(128), _rows(128), _full((1, 128))],
                    128, interpret)(degp, s2, u2, b2r)


def _tc4(degp, s3, u3, wz, bz, interpret=False):
    return _tc_call(_tc4_body,
                    [_degp_spec(), _pair(128), _rows(128), _full((128, 64)),
                     _full((1, 64))], 64, interpret)(degp, s3, u3, wz, bz)


def kernel(x, edge_index, W1, b1, W2, b2, Wm, bm, Ws, bs):
    dst3 = edge_index[1].reshape(NW, C, K)
    eidx = jnp.transpose(edge_index.reshape(2, NW, C, K), (1, 2, 0, 3))
    W2p = jnp.concatenate([W2, jnp.zeros((128, 64), W2.dtype)], axis=1)
    b2r = jnp.concatenate([b2, jnp.zeros((64,), b2.dtype)]).reshape(1, 128)
    wz = jnp.concatenate(
        [jnp.concatenate([Wm, Ws], axis=1), jnp.zeros((64, 64), Wm.dtype)],
        axis=0)
    bz = jnp.concatenate([bm, bs]).reshape(1, 2 * Wm.shape[1])

    degp = _sc_deg()(dst3)
    u1 = _tc1(degp, x, W1)
    s1 = _sc_prop(128)(eidx, u1)
    u2 = _tc2(degp, s1, u1, W2p, b1.reshape(1, -1))
    s2 = _sc_prop(128)(eidx, u2)
    u3 = _tc3(degp, s2, u2, b2r)
    s3 = _sc_prop(128)(eidx, u3)
    z = _tc4(degp, s3, u3, wz, bz)
    lat = Wm.shape[1]
    return z[:, :lat], z[:, lat:2 * lat]


# speedup vs baseline: 1.1314x; 1.1314x over previous
import functools

import jax
import jax.numpy as jnp
from jax import lax
from jax.experimental import pallas as pl
from jax.experimental.pallas import tpu as pltpu, tpu_sc as plsc

N = 10000
E = 320000
NC = 2
NS = 16
NW = NC * NS
EPT = E // NW
K = 125
C = EPT // K
NBUF = 3
RPT = 640
RLAST = N - (NS - 1) * RPT
RB = 80
DW = 128
MB = 2000


def _init_acc(buf_v, acc, s):
    for t in range(RLAST // RB):
        pltpu.sync_copy(buf_v.at[pl.ds(0, RB)],
                        acc.at[pl.ds(s * RPT + t * RB, RB)])

    @pl.when(s < NS - 1)
    def _():
        for t in range(RLAST // RB, RPT // RB):
            pltpu.sync_copy(buf_v.at[pl.ds(0, RB)],
                            acc.at[pl.ds(s * RPT + t * RB, RB)])


def _copy_out(acc, out_c, s):
    @pl.when(s < NS - 1)
    def _():
        sl = pl.ds(s * RPT, RPT)
        pltpu.sync_copy(acc.at[sl], out_c.at[sl])

    @pl.when(s == NS - 1)
    def _():
        sl = pl.ds((NS - 1) * RPT, RLAST)
        pltpu.sync_copy(acc.at[sl], out_c.at[sl])


@functools.cache
def _sc_deg():
    @functools.partial(
        pl.kernel,
        out_type=jax.ShapeDtypeStruct((NC, N, DW), jnp.float32),
        mesh=plsc.VectorSubcoreMesh(core_axis_name="c", subcore_axis_name="s"),
        scratch_types=[
            pltpu.VMEM((C, K), jnp.int32),
            pltpu.VMEM((K, DW), jnp.float32),
            pltpu.VMEM_SHARED((N, DW), jnp.float32),
        ],
        name="sc_deg",
    )
    def deg_kernel(dst_hbm, out_hbm, idx_v, buf_v, acc):
        c = lax.axis_index("c")
        s = lax.axis_index("s")
        wid = c * NS + s

        def fill(val):
            def body(r, carry):
                for q in range(DW // 16):
                    buf_v[r, pl.ds(q * 16, 16)] = jnp.full((16,), val,
                                                           jnp.float32)
                return carry
            lax.fori_loop(0, K, body, 0)

        fill(0.0)
        _init_acc(buf_v, acc, s)
        plsc.subcore_barrier()

        fill(1.0)
        pltpu.sync_copy(dst_hbm.at[wid], idx_v)

        def chunk(j, carry):
            pltpu.sync_copy(buf_v, acc.at[idx_v.at[j]], add=True)
            return carry
        lax.fori_loop(0, C, chunk, 0)
        plsc.subcore_barrier()

        _copy_out(acc, out_hbm.at[c], s)

    return deg_kernel


@functools.cache
def _sc_prop(D):
    @functools.partial(
        pl.kernel,
        out_type=jax.ShapeDtypeStruct((NC, N, D), jnp.float32),
        mesh=plsc.VectorSubcoreMesh(core_axis_name="c", subcore_axis_name="s"),
        scratch_types=(
            [pltpu.VMEM((2, K), jnp.int32) for _ in range(NBUF)]
            + [pltpu.VMEM((K, D), jnp.float32) for _ in range(NBUF)]
            + [pltpu.VMEM_SHARED((N, D), jnp.float32)]
            + [pltpu.SemaphoreType.DMA for _ in range(NBUF)]
        ),
        name=f"sc_prop{D}",
    )
    def prop(eidx_hbm, u_hbm, out_hbm, *sc):
        idxs = sc[0:NBUF]
        rows = sc[NBUF:2 * NBUF]
        acc = sc[2 * NBUF]
        sems = sc[2 * NBUF + 1:]
        c = lax.axis_index("c")
        s = lax.axis_index("s")
        wid = c * NS + s

        def zbody(r, carry):
            for q in range(D // 16):
                rows[0][r, pl.ds(q * 16, 16)] = jnp.zeros((16,), jnp.float32)
            return carry
        lax.fori_loop(0, K, zbody, 0)
        _init_acc(rows[0], acc, s)
        plsc.subcore_barrier()

        my_eidx = eidx_hbm.at[wid]

        def load_gather(j, b):
            pltpu.sync_copy(my_eidx.at[j], idxs[b])
            pltpu.async_copy(u_hbm.at[idxs[b].at[0]], rows[b], sems[b])

        def drain_scatter(b):
            pltpu.make_async_copy(u_hbm.at[idxs[b].at[0]], rows[b],
                                  sems[b]).wait()
            pltpu.sync_copy(rows[b], acc.at[idxs[b].at[1]], add=True)

        for b in range(NBUF):
            load_gather(b, b)

        def quad(t, carry):
            base = NBUF * t
            for b in range(NBUF):
                drain_scatter(b)

                @pl.when(base + b + NBUF < C)
                def _(b=b):
                    load_gather(base + b + NBUF, b)
            return carry
        lax.fori_loop(0, C // NBUF, quad, 0)
        for b in range(C % NBUF):
            drain_scatter(b)
        plsc.subcore_barrier()

        _copy_out(acc, out_hbm.at[c], s)

    return prop


def _dinv(degp_ref):
    deg = degp_ref[0, :, 0:1] + degp_ref[1, :, 0:1] + 1.0
    return lax.rsqrt(deg)


def _tc1_body(degp_ref, x_ref, w1_ref, u1_ref):
    h = jnp.dot(x_ref[...], w1_ref[...], preferred_element_type=jnp.float32)
    u1_ref[...] = _dinv(degp_ref) * h


def _tc2_body(degp_ref, s1_ref, u1_ref, w2_ref, b1_ref, u2_ref):
    dinv = _dinv(degp_ref)
    h1 = dinv * (s1_ref[0] + s1_ref[1] + u1_ref[...]) + b1_ref[...]
    h1 = jnp.maximum(h1, 0.0)
    u2_ref[...] = dinv * jnp.dot(h1, w2_ref[...], preferred_element_type=jnp.float32)


def _tc3_body(degp_ref, s2_ref, u2_ref, b2_ref, u3_ref):
    dinv = _dinv(degp_ref)
    h2 = dinv * (s2_ref[0] + s2_ref[1] + u2_ref[...]) + b2_ref[...]
    u3_ref[...] = dinv * jnp.maximum(h2, 0.0)


def _tc4_body(degp_ref, s3_ref, u3_ref, wz_ref, bz_ref, z_ref):
    p = _dinv(degp_ref) * (s3_ref[0] + s3_ref[1] + u3_ref[...])
    z_ref[...] = jnp.dot(p, wz_ref[...], preferred_element_type=jnp.float32) + bz_ref[...]


def _degp_spec():
    return pl.BlockSpec((2, MB, DW), lambda i: (0, i, 0))


def _rows(d):
    return pl.BlockSpec((MB, d), lambda i: (i, 0))


def _pair(d):
    return pl.BlockSpec((2, MB, d), lambda i: (0, i, 0))


def _full(shape):
    return pl.BlockSpec(shape, lambda i: tuple(0 for _ in shape))


def _tc_call(body, in_specs, out_d, interpret=False):
    return pl.pallas_call(
        body,
        grid=(N // MB,),
        in_specs=in_specs,
        out_specs=_rows(out_d),
        out_shape=jax.ShapeDtypeStruct((N, out_d), jnp.float32),
        interpret=interpret,
    )


def _tc1(degp, x, W1, interpret=False):
    return _tc_call(_tc1_body, [_degp_spec(), _rows(128), _full((128, 128))],
                    128, interpret)(degp, x, W1)


def _tc2(degp, s1, u1, W2p, b1r, interpret=False):
    return _tc_call(_tc2_body,
                    [_degp_spec(), _pair(128), _rows(128), _full((128, 128)),
                     _full((1, 128))], 128, interpret)(degp, s1, u1, W2p, b1r)


def _tc3(degp, s2, u2, b2r, interpret=False):
    return _tc_call(_tc3_body,
                    [_degp_spec(), _pair(128), _rows(128), _full((1, 128))],
                    128, interpret)(degp, s2, u2, b2r)


def _tc4(degp, s3, u3, wz, bz, interpret=False):
    return _tc_call(_tc4_body,
                    [_degp_spec(), _pair(128), _rows(128), _full((128, 64)),
                     _full((1, 64))], 64, interpret)(degp, s3, u3, wz, bz)


def kernel(x, edge_index, W1, b1, W2, b2, Wm, bm, Ws, bs):
    dst3 = edge_index[1].reshape(NW, C, K)
    eidx = jnp.transpose(edge_index.reshape(2, NW, C, K), (1, 2, 0, 3))
    W2p = jnp.concatenate([W2, jnp.zeros((128, 64), W2.dtype)], axis=1)
    b2r = jnp.concatenate([b2, jnp.zeros((64,), b2.dtype)]).reshape(1, 128)
    wz = jnp.concatenate(
        [jnp.concatenate([Wm, Ws], axis=1), jnp.zeros((64, 64), Wm.dtype)],
        axis=0)
    bz = jnp.concatenate([bm, bs]).reshape(1, 2 * Wm.shape[1])

    degp = _sc_deg()(dst3)
    u1 = _tc1(degp, x, W1)
    s1 = _sc_prop(128)(eidx, u1)
    u2 = _tc2(degp, s1, u1, W2p, b1.reshape(1, -1))
    s2 = _sc_prop(128)(eidx, u2)
    u3 = _tc3(degp, s2, u2, b2r)
    s3 = _sc_prop(128)(eidx, u3)
    z = _tc4(degp, s3, u3, wz, bz)
    lat = Wm.shape[1]
    return z[:, :lat], z[:, lat:2 * lat]


# speedup vs baseline: 1.2167x; 1.0754x over previous
import functools

import jax
import jax.numpy as jnp
from jax import lax
from jax.experimental import pallas as pl
from jax.experimental.pallas import tpu as pltpu, tpu_sc as plsc

N = 10000
E = 320000
NC = 2
NS = 16
NW = NC * NS
EPT = E // NW
K = 125
C = EPT // K
NBUF = 2
RPT = 640
RLAST = N - (NS - 1) * RPT
RB = 80
MB = 2000


def _init_acc(buf_v, acc, s):
    for t in range(RLAST // RB):
        pltpu.sync_copy(buf_v.at[pl.ds(0, RB)],
                        acc.at[pl.ds(s * RPT + t * RB, RB)])

    @pl.when(s < NS - 1)
    def _():
        for t in range(RLAST // RB, RPT // RB):
            pltpu.sync_copy(buf_v.at[pl.ds(0, RB)],
                            acc.at[pl.ds(s * RPT + t * RB, RB)])


def _copy_out(acc, out_c, s):
    @pl.when(s < NS - 1)
    def _():
        sl = pl.ds(s * RPT, RPT)
        pltpu.sync_copy(acc.at[sl], out_c.at[sl])

    @pl.when(s == NS - 1)
    def _():
        sl = pl.ds((NS - 1) * RPT, RLAST)
        pltpu.sync_copy(acc.at[sl], out_c.at[sl])


@functools.cache
def _sc_deg():
    NP = 10240
    CPT = NP // NS

    @functools.partial(
        pl.kernel,
        out_type=jax.ShapeDtypeStruct((NC, NP), jnp.float32),
        mesh=plsc.VectorSubcoreMesh(core_axis_name="c", subcore_axis_name="s"),
        scratch_types=[
            pltpu.VMEM((EPT,), jnp.int32),
            pltpu.VMEM((NP,), jnp.float32),
            pltpu.VMEM((NS, CPT), jnp.float32),
            pltpu.VMEM((CPT,), jnp.float32),
            pltpu.VMEM_SHARED((NS, NP), jnp.float32),
        ],
        compiler_params=pltpu.CompilerParams(needs_layout_passes=False),
        name="sc_deg",
    )
    def deg_kernel(dst_hbm, out_hbm, idx_v, hist, stage, sumbuf, acc):
        c = lax.axis_index("c")
        s = lax.axis_index("s")
        wid = c * NS + s

        def zero(r, carry):
            hist[pl.ds(16 * r, 16)] = jnp.zeros((16,), jnp.float32)
            return carry
        lax.fori_loop(0, NP // 16, zero, 0)

        pltpu.sync_copy(dst_hbm.at[pl.ds(wid * EPT, EPT)], idx_v)
        ones16 = jnp.full((16,), 1.0, jnp.float32)

        def scat(i, carry):
            v = idx_v[pl.ds(16 * i, 16)]
            plsc.addupdate_scatter(hist, [v], ones16)
            return carry
        lax.fori_loop(0, EPT // 16, scat, 0)

        pltpu.sync_copy(hist, acc.at[s])
        plsc.subcore_barrier()

        pltpu.sync_copy(acc.at[:, pl.ds(s * CPT, CPT)], stage)

        def colsum(i, carry):
            v = stage[0, pl.ds(16 * i, 16)]
            for r in range(1, NS):
                v = v + stage[r, pl.ds(16 * i, 16)]
            sumbuf[pl.ds(16 * i, 16)] = v
            return carry
        lax.fori_loop(0, CPT // 16, colsum, 0)

        pltpu.sync_copy(sumbuf, out_hbm.at[c].at[pl.ds(s * CPT, CPT)])

    return deg_kernel


@functools.cache
def _sc_prop(D):
    @functools.partial(
        pl.kernel,
        out_type=jax.ShapeDtypeStruct((NC, N, D), jnp.float32),
        mesh=plsc.VectorSubcoreMesh(core_axis_name="c", subcore_axis_name="s"),
        scratch_types=(
            [pltpu.VMEM((2, K), jnp.int32) for _ in range(NBUF)]
            + [pltpu.VMEM((K, D), jnp.float32) for _ in range(NBUF)]
            + [pltpu.VMEM_SHARED((N, D), jnp.float32)]
            + [pltpu.SemaphoreType.DMA for _ in range(NBUF)]
        ),
        name=f"sc_prop{D}",
    )
    def prop(eidx_hbm, u_hbm, out_hbm, *sc):
        idxs = sc[0:NBUF]
        rows = sc[NBUF:2 * NBUF]
        acc = sc[2 * NBUF]
        sems = sc[2 * NBUF + 1:]
        c = lax.axis_index("c")
        s = lax.axis_index("s")
        wid = c * NS + s

        def zbody(r, carry):
            for q in range(D // 16):
                rows[0][r, pl.ds(q * 16, 16)] = jnp.zeros((16,), jnp.float32)
            return carry
        lax.fori_loop(0, K, zbody, 0)
        _init_acc(rows[0], acc, s)
        plsc.subcore_barrier()

        my_eidx = eidx_hbm.at[wid]

        def load_gather(j, b):
            pltpu.sync_copy(my_eidx.at[j], idxs[b])
            pltpu.async_copy(u_hbm.at[idxs[b].at[0]], rows[b], sems[b])

        def drain_scatter(b):
            pltpu.make_async_copy(u_hbm.at[idxs[b].at[0]], rows[b],
                                  sems[b]).wait()
            pltpu.sync_copy(rows[b], acc.at[idxs[b].at[1]], add=True)

        for b in range(NBUF):
            load_gather(b, b)

        def quad(t, carry):
            base = NBUF * t
            for b in range(NBUF):
                drain_scatter(b)

                @pl.when(base + b + NBUF < C)
                def _(b=b):
                    load_gather(base + b + NBUF, b)
            return carry
        lax.fori_loop(0, C // NBUF, quad, 0)
        for b in range(C % NBUF):
            drain_scatter(b)
        plsc.subcore_barrier()

        _copy_out(acc, out_hbm.at[c], s)

    return prop


def _dinv(degp_ref):
    deg = degp_ref[0] + degp_ref[1] + 1.0
    return lax.rsqrt(deg)


def _tc1_body(degp_ref, x_ref, w1_ref, u1_ref):
    h = jnp.dot(x_ref[...], w1_ref[...], preferred_element_type=jnp.float32)
    u1_ref[...] = _dinv(degp_ref) * h


def _tc2_body(degp_ref, s1_ref, u1_ref, w2_ref, b1_ref, u2_ref):
    dinv = _dinv(degp_ref)
    h1 = dinv * (s1_ref[0] + s1_ref[1] + u1_ref[...]) + b1_ref[...]
    h1 = jnp.maximum(h1, 0.0)
    u2_ref[...] = dinv * jnp.dot(h1, w2_ref[...], preferred_element_type=jnp.float32)


def _tc3_body(degp_ref, s2_ref, u2_ref, b2_ref, u3_ref):
    dinv = _dinv(degp_ref)
    h2 = dinv * (s2_ref[0] + s2_ref[1] + u2_ref[...]) + b2_ref[...]
    u3_ref[...] = dinv * jnp.maximum(h2, 0.0)


def _tc4_body(degp_ref, s3_ref, u3_ref, wz_ref, bz_ref, z_ref):
    p = _dinv(degp_ref) * (s3_ref[0] + s3_ref[1] + u3_ref[...])
    z_ref[...] = jnp.dot(p, wz_ref[...], preferred_element_type=jnp.float32) + bz_ref[...]


def _degp_spec():
    return pl.BlockSpec((2, MB, 1), lambda i: (0, i, 0))


def _rows(d):
    return pl.BlockSpec((MB, d), lambda i: (i, 0))


def _pair(d):
    return pl.BlockSpec((2, MB, d), lambda i: (0, i, 0))


def _full(shape):
    return pl.BlockSpec(shape, lambda i: tuple(0 for _ in shape))


def _tc_call(body, in_specs, out_d, interpret=False):
    return pl.pallas_call(
        body,
        grid=(N // MB,),
        in_specs=in_specs,
        out_specs=_rows(out_d),
        out_shape=jax.ShapeDtypeStruct((N, out_d), jnp.float32),
        interpret=interpret,
    )


def _tc1(degp, x, W1, interpret=False):
    return _tc_call(_tc1_body, [_degp_spec(), _rows(128), _full((128, 128))],
                    128, interpret)(degp, x, W1)


def _tc2(degp, s1, u1, W2p, b1r, interpret=False):
    return _tc_call(_tc2_body,
                    [_degp_spec(), _pair(128), _rows(128), _full((128, 128)),
                     _full((1, 128))], 128, interpret)(degp, s1, u1, W2p, b1r)


def _tc3(degp, s2, u2, b2r, interpret=False):
    return _tc_call(_tc3_body,
                    [_degp_spec(), _pair(128), _rows(128), _full((1, 128))],
                    128, interpret)(degp, s2, u2, b2r)


def _tc4(degp, s3, u3, wz, bz, interpret=False):
    return _tc_call(_tc4_body,
                    [_degp_spec(), _pair(128), _rows(128), _full((128, 64)),
                     _full((1, 64))], 64, interpret)(degp, s3, u3, wz, bz)


def kernel(x, edge_index, W1, b1, W2, b2, Wm, bm, Ws, bs):
    eidx = jnp.transpose(edge_index.reshape(2, NW, C, K), (1, 2, 0, 3))
    W2p = jnp.concatenate([W2, jnp.zeros((128, 64), W2.dtype)], axis=1)
    b2r = jnp.concatenate([b2, jnp.zeros((64,), b2.dtype)]).reshape(1, 128)
    wz = jnp.concatenate(
        [jnp.concatenate([Wm, Ws], axis=1), jnp.zeros((64, 64), Wm.dtype)],
        axis=0)
    bz = jnp.concatenate([bm, bs]).reshape(1, 2 * Wm.shape[1])

    degp = _sc_deg()(edge_index[1])[:, :N].reshape(2, N, 1)
    u1 = _tc1(degp, x, W1)
    s1 = _sc_prop(128)(eidx, u1)
    u2 = _tc2(degp, s1, u1, W2p, b1.reshape(1, -1))
    s2 = _sc_prop(128)(eidx, u2)
    u3 = _tc3(degp, s2, u2, b2r)
    s3 = _sc_prop(128)(eidx, u3)
    z = _tc4(degp, s3, u3, wz, bz)
    lat = Wm.shape[1]
    return z[:, :lat], z[:, lat:2 * lat]


# speedup vs baseline: 1.3353x; 1.0975x over previous
import functools

import jax
import jax.numpy as jnp
from jax import lax
from jax.experimental import pallas as pl
from jax.experimental.pallas import tpu as pltpu, tpu_sc as plsc

N = 10000
E = 320000
NC = 2
NS = 16
NW = NC * NS
EPT = E // NW
K = 125
C = EPT // K
NBUF = 2
RPT = 640
RLAST = N - (NS - 1) * RPT
RB = 80
MB = 2000


def _init_acc(buf_v, acc, s):
    for t in range(RLAST // RB):
        pltpu.sync_copy(buf_v.at[pl.ds(0, RB)],
                        acc.at[pl.ds(s * RPT + t * RB, RB)])

    @pl.when(s < NS - 1)
    def _():
        for t in range(RLAST // RB, RPT // RB):
            pltpu.sync_copy(buf_v.at[pl.ds(0, RB)],
                            acc.at[pl.ds(s * RPT + t * RB, RB)])


def _copy_out(acc, out_c, s):
    @pl.when(s < NS - 1)
    def _():
        sl = pl.ds(s * RPT, RPT)
        pltpu.sync_copy(acc.at[sl], out_c.at[sl])

    @pl.when(s == NS - 1)
    def _():
        sl = pl.ds((NS - 1) * RPT, RLAST)
        pltpu.sync_copy(acc.at[sl], out_c.at[sl])


@functools.cache
def _sc_deg():
    NP = 10240
    CPT = NP // NS

    @functools.partial(
        pl.kernel,
        out_type=jax.ShapeDtypeStruct((NC, NP), jnp.float32),
        mesh=plsc.VectorSubcoreMesh(core_axis_name="c", subcore_axis_name="s"),
        scratch_types=[
            pltpu.VMEM((EPT,), jnp.int32),
            pltpu.VMEM((NP,), jnp.float32),
            pltpu.VMEM((NS, CPT), jnp.float32),
            pltpu.VMEM((CPT,), jnp.float32),
            pltpu.VMEM_SHARED((NS, NP), jnp.float32),
        ],
        compiler_params=pltpu.CompilerParams(needs_layout_passes=False),
        name="sc_deg",
    )
    def deg_kernel(dst_hbm, out_hbm, idx_v, hist, stage, sumbuf, acc):
        c = lax.axis_index("c")
        s = lax.axis_index("s")
        wid = c * NS + s

        def zero(r, carry):
            hist[pl.ds(16 * r, 16)] = jnp.zeros((16,), jnp.float32)
            return carry
        lax.fori_loop(0, NP // 16, zero, 0)

        pltpu.sync_copy(dst_hbm.at[pl.ds(wid * EPT, EPT)], idx_v)
        ones16 = jnp.full((16,), 1.0, jnp.float32)

        def scat(i, carry):
            v = idx_v[pl.ds(16 * i, 16)]
            plsc.addupdate_scatter(hist, [v], ones16)
            return carry
        lax.fori_loop(0, EPT // 16, scat, 0)

        pltpu.sync_copy(hist, acc.at[s])
        plsc.subcore_barrier()

        pltpu.sync_copy(acc.at[:, pl.ds(s * CPT, CPT)], stage)

        def colsum(i, carry):
            v = stage[0, pl.ds(16 * i, 16)]
            for r in range(1, NS):
                v = v + stage[r, pl.ds(16 * i, 16)]
            sumbuf[pl.ds(16 * i, 16)] = v
            return carry
        lax.fori_loop(0, CPT // 16, colsum, 0)

        pltpu.sync_copy(sumbuf, out_hbm.at[c].at[pl.ds(s * CPT, CPT)])

    return deg_kernel


@functools.cache
def _sc_prop(D):
    Q = C // 4

    @functools.partial(
        pl.kernel,
        out_type=jax.ShapeDtypeStruct((NC, N, D), jnp.float32),
        mesh=plsc.VectorSubcoreMesh(core_axis_name="c", subcore_axis_name="s"),
        scratch_types=[
            pltpu.VMEM((2, 2, K), jnp.int32),
            pltpu.VMEM((2, 2, K), jnp.int32),
            pltpu.VMEM((K, D), jnp.float32),
            pltpu.VMEM((K, D), jnp.float32),
            pltpu.VMEM_SHARED((N, D), jnp.float32),
            pltpu.SemaphoreType.DMA,
            pltpu.SemaphoreType.DMA,
            pltpu.SemaphoreType.DMA,
            pltpu.SemaphoreType.DMA,
        ],
        name=f"sc_prop{D}",
    )
    def prop(eidx_hbm, u_hbm, out_hbm, idxA, idxB, rows0, rows1, acc,
             sem0, sem1, isemA, isemB):
        c = lax.axis_index("c")
        s = lax.axis_index("s")
        wid = c * NS + s

        def zbody(r, carry):
            for q in range(D // 16):
                rows0[r, pl.ds(q * 16, 16)] = jnp.zeros((16,), jnp.float32)
            return carry
        lax.fori_loop(0, K, zbody, 0)
        _init_acc(rows0, acc, s)
        plsc.subcore_barrier()

        my = eidx_hbm.at[wid]

        def gather(idx_sl, rows, sem):
            pltpu.async_copy(u_hbm.at[idx_sl], rows, sem)

        def drain_scatter(idx2, e, rows, sem):
            pltpu.make_async_copy(u_hbm.at[idx2.at[e].at[0]], rows, sem).wait()
            pltpu.sync_copy(rows, acc.at[idx2.at[e].at[1]], add=True)

        pltpu.sync_copy(my.at[0], idxA)
        gather(idxA.at[0].at[0], rows0, sem0)
        gather(idxA.at[1].at[0], rows1, sem1)
        pltpu.async_copy(my.at[1], idxB, isemB)

        def quad(q, carry):
            pltpu.make_async_copy(my.at[2 * q + 1], idxB, isemB).wait()
            drain_scatter(idxA, 0, rows0, sem0)
            gather(idxB.at[0].at[0], rows0, sem0)
            drain_scatter(idxA, 1, rows1, sem1)

            @pl.when(q + 1 < Q)
            def _():
                pltpu.async_copy(my.at[2 * q + 2], idxA, isemA)
            gather(idxB.at[1].at[0], rows1, sem1)
            drain_scatter(idxB, 0, rows0, sem0)

            @pl.when(q + 1 < Q)
            def _():
                pltpu.make_async_copy(my.at[2 * q + 2], idxA, isemA).wait()
                gather(idxA.at[0].at[0], rows0, sem0)
            drain_scatter(idxB, 1, rows1, sem1)

            @pl.when(q + 1 < Q)
            def _():
                pltpu.async_copy(my.at[2 * q + 3], idxB, isemB)
                gather(idxA.at[1].at[0], rows1, sem1)
            return carry
        lax.fori_loop(0, Q, quad, 0)
        plsc.subcore_barrier()

        _copy_out(acc, out_hbm.at[c], s)

    return prop


def _dinv(degp_ref):
    deg = degp_ref[0] + degp_ref[1] + 1.0
    return lax.rsqrt(deg)


def _tc1_body(degp_ref, x_ref, w1_ref, u1_ref):
    h = jnp.dot(x_ref[...], w1_ref[...], preferred_element_type=jnp.float32)
    u1_ref[...] = _dinv(degp_ref) * h


def _tc2_body(degp_ref, s1_ref, u1_ref, w2_ref, b1_ref, u2_ref):
    dinv = _dinv(degp_ref)
    h1 = dinv * (s1_ref[0] + s1_ref[1] + u1_ref[...]) + b1_ref[...]
    h1 = jnp.maximum(h1, 0.0)
    u2_ref[...] = dinv * jnp.dot(h1, w2_ref[...], preferred_element_type=jnp.float32)


def _tc3_body(degp_ref, s2_ref, u2_ref, b2_ref, u3_ref):
    dinv = _dinv(degp_ref)
    h2 = dinv * (s2_ref[0] + s2_ref[1] + u2_ref[...]) + b2_ref[...]
    u3_ref[...] = dinv * jnp.maximum(h2, 0.0)


def _tc4_body(degp_ref, s3_ref, u3_ref, wz_ref, bz_ref, z_ref):
    p = _dinv(degp_ref) * (s3_ref[0] + s3_ref[1] + u3_ref[...])
    z_ref[...] = jnp.dot(p, wz_ref[...], preferred_element_type=jnp.float32) + bz_ref[...]


def _degp_spec():
    return pl.BlockSpec((2, MB, 1), lambda i: (0, i, 0))


def _rows(d):
    return pl.BlockSpec((MB, d), lambda i: (i, 0))


def _pair(d):
    return pl.BlockSpec((2, MB, d), lambda i: (0, i, 0))


def _full(shape):
    return pl.BlockSpec(shape, lambda i: tuple(0 for _ in shape))


def _tc_call(body, in_specs, out_d, interpret=False):
    return pl.pallas_call(
        body,
        grid=(N // MB,),
        in_specs=in_specs,
        out_specs=_rows(out_d),
        out_shape=jax.ShapeDtypeStruct((N, out_d), jnp.float32),
        interpret=interpret,
    )


def _tc1(degp, x, W1, interpret=False):
    return _tc_call(_tc1_body, [_degp_spec(), _rows(128), _full((128, 128))],
                    128, interpret)(degp, x, W1)


def _tc2(degp, s1, u1, W2p, b1r, interpret=False):
    return _tc_call(_tc2_body,
                    [_degp_spec(), _pair(128), _rows(128), _full((128, 128)),
                     _full((1, 128))], 128, interpret)(degp, s1, u1, W2p, b1r)


def _tc3(degp, s2, u2, b2r, interpret=False):
    return _tc_call(_tc3_body,
                    [_degp_spec(), _pair(128), _rows(128), _full((1, 128))],
                    128, interpret)(degp, s2, u2, b2r)


def _tc4(degp, s3, u3, wz, bz, interpret=False):
    return _tc_call(_tc4_body,
                    [_degp_spec(), _pair(128), _rows(128), _full((128, 64)),
                     _full((1, 64))], 64, interpret)(degp, s3, u3, wz, bz)


def kernel(x, edge_index, W1, b1, W2, b2, Wm, bm, Ws, bs):
    eidx = jnp.transpose(edge_index.reshape(2, NW, C, K),
                         (1, 2, 0, 3)).reshape(NW, C // 2, 2, 2, K)
    W2p = jnp.concatenate([W2, jnp.zeros((128, 64), W2.dtype)], axis=1)
    b2r = jnp.concatenate([b2, jnp.zeros((64,), b2.dtype)]).reshape(1, 128)
    wz = jnp.concatenate(
        [jnp.concatenate([Wm, Ws], axis=1), jnp.zeros((64, 64), Wm.dtype)],
        axis=0)
    bz = jnp.concatenate([bm, bs]).reshape(1, 2 * Wm.shape[1])

    degp = _sc_deg()(edge_index[1])[:, :N].reshape(2, N, 1)
    u1 = _tc1(degp, x, W1)
    s1 = _sc_prop(128)(eidx, u1)
    u2 = _tc2(degp, s1, u1, W2p, b1.reshape(1, -1))
    s2 = _sc_prop(128)(eidx, u2)
    u3 = _tc3(degp, s2, u2, b2r)
    s3 = _sc_prop(128)(eidx, u3)
    z = _tc4(degp, s3, u3, wz, bz)
    lat = Wm.shape[1]
    return z[:, :lat], z[:, lat:2 * lat]
